# Initial kernel scaffold; baseline (speedup 1.0000x reference)
import jax
import jax.numpy as jnp
from jax.experimental import pallas as pl


def kernel(queries, keys):
    raise NotImplementedError("write your pallas kernel here")


# speedup vs baseline: 1.6650x; 1.6650x over previous
import functools

import jax
import jax.numpy as jnp
from jax.experimental import pallas as pl
from jax.experimental.pallas import tpu as pltpu
from jax.experimental.pallas import tpu_sc as plsc

TOP_K = 10
GAMMA = 0.7
TEMPERATURE = 0.5
EPS = 1e-8
NEG = -1e30
POS = 1e30

Q = 1024
D = 128
K = 100000
KT = 2048
NT = (K + KT - 1) // KT
KSLOT = 16


def _topk_body(q_ref, k_ref, w_ref, idx_ref, vacc, iacc):
    t = pl.program_id(0)

    @pl.when(t == 0)
    def _init():
        lane16 = jax.lax.broadcasted_iota(jnp.int32, (Q, KSLOT), 1)
        vacc[...] = jnp.where(lane16 < TOP_K, NEG, POS).astype(jnp.float32)
        iacc[...] = jnp.zeros((Q, KSLOT), jnp.int32)

    q = q_ref[...]
    qn = q / (jnp.sqrt(jnp.sum(q * q, axis=1, keepdims=True)) + EPS)
    k = k_ref[...]
    kn = k / (jnp.sqrt(jnp.sum(k * k, axis=1, keepdims=True)) + EPS)

    sim = jax.lax.dot_general(qn, kn, (((1,), (1,)), ((), ())),
                              preferred_element_type=jnp.float32)

    col = jax.lax.broadcasted_iota(jnp.int32, (Q, KT), 1)
    gcol = col + t * KT
    sim = jnp.where(gcol < K, sim, NEG)

    lane16 = jax.lax.broadcasted_iota(jnp.int32, (Q, KSLOT), 1)
    for _ in range(TOP_K):
        m = jnp.max(sim, axis=1)
        am = jnp.min(jnp.where(sim == m[:, None], col, KT + 1), axis=1)
        sim = jnp.where(col == am[:, None], NEG, sim)

        rv = vacc[...]
        rmin = jnp.min(rv, axis=1)
        lmin = jnp.min(jnp.where(rv == rmin[:, None], lane16, KSLOT + 1),
                       axis=1)
        better = m > rmin
        sel = (lane16 == lmin[:, None]) & better[:, None]
        vacc[...] = jnp.where(sel, m[:, None], rv)
        iacc[...] = jnp.where(sel, (am + t * KT)[:, None], iacc[...])

    @pl.when(t == NT - 1)
    def _final():
        v = vacc[...]
        v = jnp.where(lane16 < TOP_K, v, NEG)
        mx = jnp.max(v, axis=1, keepdims=True)
        e = jnp.exp((v - mx) / TEMPERATURE)
        w_ref[...] = e / jnp.sum(e, axis=1, keepdims=True)
        idx_ref[...] = iacc[...]


def _topk_weights(queries, keys):
    return pl.pallas_call(
        _topk_body,
        grid=(NT,),
        in_specs=[
            pl.BlockSpec((Q, D), lambda t: (0, 0)),
            pl.BlockSpec((KT, D), lambda t: (t, 0)),
        ],
        out_specs=[
            pl.BlockSpec((Q, KSLOT), lambda t: (0, 0)),
            pl.BlockSpec((Q, KSLOT), lambda t: (0, 0)),
        ],
        out_shape=[
            jax.ShapeDtypeStruct((Q, KSLOT), jnp.float32),
            jax.ShapeDtypeStruct((Q, KSLOT), jnp.int32),
        ],
        scratch_shapes=[
            pltpu.VMEM((Q, KSLOT), jnp.float32),
            pltpu.VMEM((Q, KSLOT), jnp.int32),
        ],
    )(queries, keys)


_B = Q * KSLOT
_NW = 32
_BPW = _B // _NW
_CH = 128


def _sc_gather(keys, idx_flat):
    mesh = plsc.VectorSubcoreMesh(core_axis_name="c", subcore_axis_name="s")

    @functools.partial(
        pl.kernel,
        out_type=jax.ShapeDtypeStruct((_B, D), jnp.float32),
        mesh=mesh,
        scratch_types=[
            pltpu.VMEM((_BPW,), jnp.int32),
            pltpu.VMEM((_BPW, D), jnp.float32),
            pltpu.SemaphoreType.DMA,
        ],
    )
    def body(keys_hbm, idx_hbm, out_hbm, idx_v, rows_v, sem):
        wid = jax.lax.axis_index("s") * 2 + jax.lax.axis_index("c")
        base = wid * _BPW
        pltpu.sync_copy(idx_hbm.at[pl.ds(base, _BPW)], idx_v)
        copies = []
        for c in range(_BPW // _CH):
            copies.append(pltpu.async_copy(
                keys_hbm.at[idx_v.at[pl.ds(c * _CH, _CH)]],
                rows_v.at[pl.ds(c * _CH, _CH)], sem))
        for cp in copies:
            cp.wait()
        pltpu.sync_copy(rows_v, out_hbm.at[pl.ds(base, _BPW)])

    return body(keys, idx_flat)


def _combine_body(q_ref, w_ref, rows_ref, o_ref):
    q = q_ref[...]
    w = w_ref[...]
    rows = rows_ref[...]
    agg = jnp.sum(rows * w[:, :, None], axis=1)
    o_ref[...] = GAMMA * q + (1.0 - GAMMA) * agg


def _combine(queries, weights, rows):
    return pl.pallas_call(
        _combine_body,
        out_shape=jax.ShapeDtypeStruct((Q, D), jnp.float32),
    )(queries, weights, rows)


def kernel(queries, keys):
    weights, idx = _topk_weights(queries, keys)
    rows = _sc_gather(keys, idx.reshape(_B))
    return _combine(queries, weights, rows.reshape(Q, KSLOT, D))


# speedup vs baseline: 2.4859x; 1.4930x over previous
import functools

import jax
import jax.numpy as jnp
from jax.experimental import pallas as pl
from jax.experimental.pallas import tpu as pltpu
from jax.experimental.pallas import tpu_sc as plsc

TOP_K = 10
GAMMA = 0.7
TEMPERATURE = 0.5
EPS = 1e-8
NEG = -1e30
POS = 1e30

Q = 1024
D = 128
K = 100000
KT = 2048
NT = (K + KT - 1) // KT
KSLOT = 16


def _topk_body(q_ref, k_ref, w_ref, idx_ref, vacc, iacc, sim_ref):
    t = pl.program_id(0)

    @pl.when(t == 0)
    def _init():
        lane16 = jax.lax.broadcasted_iota(jnp.int32, (Q, KSLOT), 1)
        vacc[...] = jnp.where(lane16 < TOP_K, NEG, POS).astype(jnp.float32)
        iacc[...] = jnp.zeros((Q, KSLOT), jnp.int32)

    q = q_ref[...]
    qn = q / (jnp.sqrt(jnp.sum(q * q, axis=1, keepdims=True)) + EPS)
    k = k_ref[...]
    kn = k / (jnp.sqrt(jnp.sum(k * k, axis=1, keepdims=True)) + EPS)

    sim = jax.lax.dot_general(qn, kn, (((1,), (1,)), ((), ())),
                              preferred_element_type=jnp.float32)

    col = jax.lax.broadcasted_iota(jnp.int32, (Q, KT), 1)
    gcol = col + t * KT
    sim_ref[...] = jnp.where(gcol < K, sim, NEG)

    lane16 = jax.lax.broadcasted_iota(jnp.int32, (Q, KSLOT), 1)

    def _body(state):
        i, _ = state
        s = sim_ref[...]
        m = jnp.max(s, axis=1)
        am = jnp.min(jnp.where(s == m[:, None], col, KT + 1), axis=1)

        rv = vacc[...]
        rmin = jnp.min(rv, axis=1)
        better = m > rmin
        any_better = jnp.any(better)

        sim_ref[...] = jnp.where(col == am[:, None], NEG, s)
        lmin = jnp.min(jnp.where(rv == rmin[:, None], lane16, KSLOT + 1),
                       axis=1)
        sel = (lane16 == lmin[:, None]) & better[:, None]
        vacc[...] = jnp.where(sel, m[:, None], rv)
        iacc[...] = jnp.where(sel, (am + t * KT)[:, None], iacc[...])
        return i + 1, any_better

    def _cond(state):
        i, any_better = state
        return (i < TOP_K) & any_better

    jax.lax.while_loop(_cond, _body, (0, True))

    @pl.when(t == NT - 1)
    def _final():
        v = vacc[...]
        v = jnp.where(lane16 < TOP_K, v, NEG)
        mx = jnp.max(v, axis=1, keepdims=True)
        e = jnp.exp((v - mx) / TEMPERATURE)
        w_ref[...] = e / jnp.sum(e, axis=1, keepdims=True)
        idx_ref[...] = iacc[...]


def _topk_weights(queries, keys):
    return pl.pallas_call(
        _topk_body,
        grid=(NT,),
        in_specs=[
            pl.BlockSpec((Q, D), lambda t: (0, 0)),
            pl.BlockSpec((KT, D), lambda t: (t, 0)),
        ],
        out_specs=[
            pl.BlockSpec((Q, KSLOT), lambda t: (0, 0)),
            pl.BlockSpec((Q, KSLOT), lambda t: (0, 0)),
        ],
        out_shape=[
            jax.ShapeDtypeStruct((Q, KSLOT), jnp.float32),
            jax.ShapeDtypeStruct((Q, KSLOT), jnp.int32),
        ],
        scratch_shapes=[
            pltpu.VMEM((Q, KSLOT), jnp.float32),
            pltpu.VMEM((Q, KSLOT), jnp.int32),
            pltpu.VMEM((Q, KT), jnp.float32),
        ],
    )(queries, keys)


_B = Q * KSLOT
_NW = 32
_BPW = _B // _NW
_CH = 128


def _sc_gather(keys, idx_flat):
    mesh = plsc.VectorSubcoreMesh(core_axis_name="c", subcore_axis_name="s")

    @functools.partial(
        pl.kernel,
        out_type=jax.ShapeDtypeStruct((_B, D), jnp.float32),
        mesh=mesh,
        scratch_types=[
            pltpu.VMEM((_BPW,), jnp.int32),
            pltpu.VMEM((_BPW, D), jnp.float32),
            pltpu.SemaphoreType.DMA,
        ],
    )
    def body(keys_hbm, idx_hbm, out_hbm, idx_v, rows_v, sem):
        wid = jax.lax.axis_index("s") * 2 + jax.lax.axis_index("c")
        base = wid * _BPW
        pltpu.sync_copy(idx_hbm.at[pl.ds(base, _BPW)], idx_v)
        copies = []
        for c in range(_BPW // _CH):
            copies.append(pltpu.async_copy(
                keys_hbm.at[idx_v.at[pl.ds(c * _CH, _CH)]],
                rows_v.at[pl.ds(c * _CH, _CH)], sem))
        for cp in copies:
            cp.wait()
        pltpu.sync_copy(rows_v, out_hbm.at[pl.ds(base, _BPW)])

    return body(keys, idx_flat)


def _combine_body(q_ref, w_ref, rows_ref, o_ref):
    q = q_ref[...]
    w = w_ref[...]
    rows = rows_ref[...]
    agg = jnp.sum(rows * w[:, :, None], axis=1)
    o_ref[...] = GAMMA * q + (1.0 - GAMMA) * agg


def _combine(queries, weights, rows):
    return pl.pallas_call(
        _combine_body,
        out_shape=jax.ShapeDtypeStruct((Q, D), jnp.float32),
    )(queries, weights, rows)


def kernel(queries, keys):
    weights, idx = _topk_weights(queries, keys)
    rows = _sc_gather(keys, idx.reshape(_B))
    return _combine(queries, weights, rows.reshape(Q, KSLOT, D))


# speedup vs baseline: 2.6051x; 1.0480x over previous
import functools

import jax
import jax.numpy as jnp
from jax.experimental import pallas as pl
from jax.experimental.pallas import tpu as pltpu
from jax.experimental.pallas import tpu_sc as plsc

TOP_K = 10
GAMMA = 0.7
TEMPERATURE = 0.5
EPS = 1e-8
NEG = -1e30
POS = 1e30

Q = 1024
D = 128
K = 100000
KT = 2048
NT = (K + KT - 1) // KT
KSLOT = 16


def _topk_body(q_ref, k_ref, w_ref, idx_ref, vacc, iacc, sim_ref):
    t = pl.program_id(0)

    @pl.when(t == 0)
    def _init():
        lane16 = jax.lax.broadcasted_iota(jnp.int32, (Q, KSLOT), 1)
        vacc[...] = jnp.where(lane16 < TOP_K, NEG, POS).astype(jnp.float32)
        iacc[...] = jnp.zeros((Q, KSLOT), jnp.int32)

    q = q_ref[...]
    qn = q / (jnp.sqrt(jnp.sum(q * q, axis=1, keepdims=True)) + EPS)
    k = k_ref[...]
    kn = k / (jnp.sqrt(jnp.sum(k * k, axis=1, keepdims=True)) + EPS)

    sim = jax.lax.dot_general(qn, kn, (((1,), (1,)), ((), ())),
                              preferred_element_type=jnp.float32)

    col = jax.lax.broadcasted_iota(jnp.int32, (Q, KT), 1)
    gcol = col + t * KT
    sim_ref[...] = jnp.where(gcol < K, sim, NEG)

    lane16 = jax.lax.broadcasted_iota(jnp.int32, (Q, KSLOT), 1)

    m0 = jnp.max(sim_ref[...], axis=1)
    rmin0 = jnp.min(vacc[...], axis=1)

    def _body(state):
        i, m, _ = state
        s = sim_ref[...]
        am = jnp.min(jnp.where(s == m[:, None], col, KT + 1), axis=1)
        masked = jnp.where(col == am[:, None], NEG, s)
        sim_ref[...] = masked
        m_next = jnp.max(masked, axis=1)

        rv = vacc[...]
        rmin = jnp.min(rv, axis=1)
        better = m > rmin
        lmin = jnp.min(jnp.where(rv == rmin[:, None], lane16, KSLOT + 1),
                       axis=1)
        sel = (lane16 == lmin[:, None]) & better[:, None]
        nv = jnp.where(sel, m[:, None], rv)
        vacc[...] = nv
        iacc[...] = jnp.where(sel, (am + t * KT)[:, None], iacc[...])
        go = jnp.any(m_next > jnp.min(nv, axis=1))
        return i + 1, m_next, go

    def _cond(state):
        i, _, go = state
        return (i < TOP_K) & go

    jax.lax.while_loop(_cond, _body, (0, m0, jnp.any(m0 > rmin0)))

    @pl.when(t == NT - 1)
    def _final():
        v = vacc[...]
        v = jnp.where(lane16 < TOP_K, v, NEG)
        mx = jnp.max(v, axis=1, keepdims=True)
        e = jnp.exp((v - mx) / TEMPERATURE)
        w_ref[...] = e / jnp.sum(e, axis=1, keepdims=True)
        idx_ref[...] = iacc[...]


def _topk_weights(queries, keys):
    return pl.pallas_call(
        _topk_body,
        grid=(NT,),
        in_specs=[
            pl.BlockSpec((Q, D), lambda t: (0, 0)),
            pl.BlockSpec((KT, D), lambda t: (t, 0)),
        ],
        out_specs=[
            pl.BlockSpec((Q, KSLOT), lambda t: (0, 0)),
            pl.BlockSpec((Q, KSLOT), lambda t: (0, 0)),
        ],
        out_shape=[
            jax.ShapeDtypeStruct((Q, KSLOT), jnp.float32),
            jax.ShapeDtypeStruct((Q, KSLOT), jnp.int32),
        ],
        scratch_shapes=[
            pltpu.VMEM((Q, KSLOT), jnp.float32),
            pltpu.VMEM((Q, KSLOT), jnp.int32),
            pltpu.VMEM((Q, KT), jnp.float32),
        ],
    )(queries, keys)


_B = Q * KSLOT
_NW = 32
_BPW = _B // _NW
_CH = 128


def _sc_gather(keys, idx_flat):
    mesh = plsc.VectorSubcoreMesh(core_axis_name="c", subcore_axis_name="s")

    @functools.partial(
        pl.kernel,
        out_type=jax.ShapeDtypeStruct((_B, D), jnp.float32),
        mesh=mesh,
        scratch_types=[
            pltpu.VMEM((_BPW,), jnp.int32),
            pltpu.VMEM((_BPW, D), jnp.float32),
            pltpu.SemaphoreType.DMA,
        ],
    )
    def body(keys_hbm, idx_hbm, out_hbm, idx_v, rows_v, sem):
        wid = jax.lax.axis_index("s") * 2 + jax.lax.axis_index("c")
        base = wid * _BPW
        pltpu.sync_copy(idx_hbm.at[pl.ds(base, _BPW)], idx_v)
        copies = []
        for c in range(_BPW // _CH):
            copies.append(pltpu.async_copy(
                keys_hbm.at[idx_v.at[pl.ds(c * _CH, _CH)]],
                rows_v.at[pl.ds(c * _CH, _CH)], sem))
        for cp in copies:
            cp.wait()
        pltpu.sync_copy(rows_v, out_hbm.at[pl.ds(base, _BPW)])

    return body(keys, idx_flat)


def _combine_body(q_ref, w_ref, rows_ref, o_ref):
    q = q_ref[...]
    w = w_ref[...]
    rows = rows_ref[...]
    agg = jnp.sum(rows * w[:, :, None], axis=1)
    o_ref[...] = GAMMA * q + (1.0 - GAMMA) * agg


def _combine(queries, weights, rows):
    return pl.pallas_call(
        _combine_body,
        out_shape=jax.ShapeDtypeStruct((Q, D), jnp.float32),
    )(queries, weights, rows)


def kernel(queries, keys):
    weights, idx = _topk_weights(queries, keys)
    rows = _sc_gather(keys, idx.reshape(_B))
    return _combine(queries, weights, rows.reshape(Q, KSLOT, D))
